# baseline (speedup 1.0000x reference)
import functools

import jax
import jax.numpy as jnp
from jax.experimental import pallas as pl
from jax.experimental.pallas import tpu as pltpu

B, T = 16, 512
ACTION_DIM, STATE_DIM, EMBED_DIM, H_DIM = 64, 1024, 128, 512
CT = 64
NT = T // CT

_PREC = jax.lax.Precision.DEFAULT


def _mm(a, b):
    return jax.lax.dot_general(a, b, (((1,), (0,)), ((), ())),
                               preferred_element_type=jnp.float32,
                               precision=_PREC)


def _mmt(a, b):
    return jax.lax.dot_general(a, b, (((1,), (1,)), ((), ())),
                               preferred_element_type=jnp.float32,
                               precision=_PREC)


def _rnn_kernel(a_ref, idx_ref, emb_ref, w_ih_ref, w_hh_ref, b_ih_ref,
                b_hh_ref, out_ref, h_ref, z_ref):
    i = pl.program_id(0)

    @pl.when(i == 0)
    def _init():
        h_ref[...] = jnp.zeros_like(h_ref)

    @pl.when(i < NT)
    def _project():
        idx = idx_ref[...].reshape(B * CT, 1)
        iota = jax.lax.broadcasted_iota(jnp.int32, (B * CT, STATE_DIM), 1)
        onehot = (idx == iota).astype(jnp.float32)
        s_emb = _mm(onehot, emb_ref[...])
        a2d = a_ref[...].reshape(B * CT, ACTION_DIM)
        z = (_mmt(a2d, w_ih_ref[:, :ACTION_DIM])
             + _mmt(s_emb, w_ih_ref[:, ACTION_DIM:])
             + b_ih_ref[...] + b_hh_ref[...])
        z_ref[i % 2] = z.reshape(B, CT, H_DIM)

    @pl.when(i > 0)
    def _recur():
        HH = H_DIM // 2
        w00 = w_hh_ref[:HH, :HH]
        w01 = w_hh_ref[:HH, HH:]
        w10 = w_hh_ref[HH:, :HH]
        w11 = w_hh_ref[HH:, HH:]
        zb = (i - 1) % 2
        ha = h_ref[:, :HH]
        hb = h_ref[:, HH:]
        for k in range(CT):
            zk = z_ref[zb, :, k, :]
            pre0 = zk[:, :HH] + _mmt(ha, w00) + _mmt(hb, w01)
            pre1 = zk[:, HH:] + _mmt(ha, w10) + _mmt(hb, w11)
            ha = jnp.tanh(pre0)
            hb = jnp.tanh(pre1)
            out_ref[:, k, :HH] = ha
            out_ref[:, k, HH:] = hb
        h_ref[:, :HH] = ha
        h_ref[:, HH:] = hb


@jax.jit
def kernel(actions, state_indices, emb, W_ih, W_hh, b_ih, b_hh):
    idx3 = state_indices.reshape(B, T, 1).astype(jnp.int32)

    last = NT - 1
    out = pl.pallas_call(
        _rnn_kernel,
        grid=(NT + 1,),
        in_specs=[
            pl.BlockSpec((B, CT, ACTION_DIM),
                         lambda i: (0, jnp.minimum(i, last), 0)),
            pl.BlockSpec((B, CT, 1), lambda i: (0, jnp.minimum(i, last), 0)),
            pl.BlockSpec((STATE_DIM, EMBED_DIM), lambda i: (0, 0)),
            pl.BlockSpec((H_DIM, ACTION_DIM + EMBED_DIM), lambda i: (0, 0)),
            pl.BlockSpec((H_DIM, H_DIM), lambda i: (0, 0)),
            pl.BlockSpec((1, H_DIM), lambda i: (0, 0)),
            pl.BlockSpec((1, H_DIM), lambda i: (0, 0)),
        ],
        out_specs=pl.BlockSpec((B, CT, H_DIM),
                               lambda i: (0, jnp.maximum(i - 1, 0), 0)),
        out_shape=jax.ShapeDtypeStruct((B, T, H_DIM), jnp.float32),
        scratch_shapes=[pltpu.VMEM((B, H_DIM), jnp.float32),
                        pltpu.VMEM((2, B, CT, H_DIM), jnp.float32)],
    )(actions, idx3, emb, W_ih, W_hh,
      b_ih.reshape(1, H_DIM), b_hh.reshape(1, H_DIM))

    return out


# speedup vs baseline: 1.0331x; 1.0331x over previous
import functools

import jax
import jax.numpy as jnp
from jax.experimental import pallas as pl
from jax.experimental.pallas import tpu as pltpu

B, T = 16, 512
ACTION_DIM, STATE_DIM, EMBED_DIM, H_DIM = 64, 1024, 128, 512
CT = 64
NT = T // CT

_PREC = jax.lax.Precision.DEFAULT


def _mm(a, b):
    return jax.lax.dot_general(a, b, (((1,), (0,)), ((), ())),
                               preferred_element_type=jnp.float32,
                               precision=_PREC)


def _mmt(a, b):
    return jax.lax.dot_general(a, b, (((1,), (1,)), ((), ())),
                               preferred_element_type=jnp.float32,
                               precision=_PREC)


def _rnn_kernel(a_ref, idx_ref, emb_ref, w_ih_ref, w_hh_ref, b_ih_ref,
                b_hh_ref, out_ref, h_ref, z_ref):
    i = pl.program_id(0)

    @pl.when(i == 0)
    def _init():
        h_ref[...] = jnp.zeros_like(h_ref)

    @pl.when(i < NT)
    def _project():
        idx = idx_ref[...]
        iota = jax.lax.broadcasted_iota(jnp.int32, (CT * B, STATE_DIM), 1)
        onehot = (idx == iota).astype(jnp.float32)
        s_emb = _mm(onehot, emb_ref[...])
        z_ref[i % 2] = (_mmt(a_ref[...], w_ih_ref[:, :ACTION_DIM])
                        + _mmt(s_emb, w_ih_ref[:, ACTION_DIM:])
                        + b_ih_ref[...] + b_hh_ref[...])

    @pl.when(i > 0)
    def _recur():
        HH = H_DIM // 2
        w00 = w_hh_ref[:HH, :HH]
        w01 = w_hh_ref[:HH, HH:]
        w10 = w_hh_ref[HH:, :HH]
        w11 = w_hh_ref[HH:, HH:]
        zb = (i - 1) % 2
        ha = h_ref[:, :HH]
        hb = h_ref[:, HH:]
        for k in range(CT):
            zk = z_ref[zb, k * B:(k + 1) * B, :]
            pre0 = zk[:, :HH] + _mmt(ha, w00) + _mmt(hb, w01)
            pre1 = zk[:, HH:] + _mmt(ha, w10) + _mmt(hb, w11)
            ha = jnp.tanh(pre0)
            hb = jnp.tanh(pre1)
            out_ref[:, k, :HH] = ha
            out_ref[:, k, HH:] = hb
        h_ref[:, :HH] = ha
        h_ref[:, HH:] = hb


@jax.jit
def kernel(actions, state_indices, emb, W_ih, W_hh, b_ih, b_hh):
    a_tm = jnp.swapaxes(actions, 0, 1).reshape(T * B, ACTION_DIM)
    idx_tm = jnp.swapaxes(state_indices, 0, 1).reshape(T * B, 1).astype(jnp.int32)

    last = NT - 1
    out = pl.pallas_call(
        _rnn_kernel,
        grid=(NT + 1,),
        in_specs=[
            pl.BlockSpec((CT * B, ACTION_DIM), lambda i: (jnp.minimum(i, last), 0)),
            pl.BlockSpec((CT * B, 1), lambda i: (jnp.minimum(i, last), 0)),
            pl.BlockSpec((STATE_DIM, EMBED_DIM), lambda i: (0, 0)),
            pl.BlockSpec((H_DIM, ACTION_DIM + EMBED_DIM), lambda i: (0, 0)),
            pl.BlockSpec((H_DIM, H_DIM), lambda i: (0, 0)),
            pl.BlockSpec((1, H_DIM), lambda i: (0, 0)),
            pl.BlockSpec((1, H_DIM), lambda i: (0, 0)),
        ],
        out_specs=pl.BlockSpec((B, CT, H_DIM),
                               lambda i: (0, jnp.maximum(i - 1, 0), 0)),
        out_shape=jax.ShapeDtypeStruct((B, T, H_DIM), jnp.float32),
        scratch_shapes=[pltpu.VMEM((B, H_DIM), jnp.float32),
                        pltpu.VMEM((2, CT * B, H_DIM), jnp.float32)],
    )(a_tm, idx_tm, emb, W_ih, W_hh,
      b_ih.reshape(1, H_DIM), b_hh.reshape(1, H_DIM))

    return out


# speedup vs baseline: 1.1358x; 1.0994x over previous
import functools

import jax
import jax.numpy as jnp
from jax.experimental import pallas as pl
from jax.experimental.pallas import tpu as pltpu

B, T = 16, 512
ACTION_DIM, STATE_DIM, EMBED_DIM, H_DIM = 64, 1024, 128, 512
CT = 64
NT = T // CT

_PREC = jax.lax.Precision.DEFAULT


def _mm(a, b):
    return jax.lax.dot_general(a, b, (((1,), (0,)), ((), ())),
                               preferred_element_type=jnp.float32,
                               precision=_PREC)


def _mmt(a, b):
    return jax.lax.dot_general(a, b, (((1,), (1,)), ((), ())),
                               preferred_element_type=jnp.float32,
                               precision=_PREC)


def _rnn_kernel(a_ref, idx_ref, emb_ref, w_ih_ref, w_hh_ref, b_ih_ref,
                b_hh_ref, out_ref, h_ref, z_ref):
    i = pl.program_id(0)

    @pl.when(i == 0)
    def _init():
        h_ref[...] = jnp.zeros_like(h_ref)

    @pl.when(i < NT)
    def _project():
        idx = idx_ref[...]
        iota = jax.lax.broadcasted_iota(jnp.int32, (CT * B, STATE_DIM), 1)
        onehot = (idx == iota).astype(jnp.float32)
        s_emb = _mm(onehot, emb_ref[...])
        z_ref[i % 2] = (_mmt(a_ref[...], w_ih_ref[:, :ACTION_DIM])
                        + _mmt(s_emb, w_ih_ref[:, ACTION_DIM:])
                        + b_ih_ref[...] + b_hh_ref[...])

    @pl.when(i > 0)
    def _recur():
        HH = H_DIM // 2
        w00 = w_hh_ref[:HH, :HH]
        w01 = w_hh_ref[HH:, :HH]
        w10 = w_hh_ref[:HH, HH:]
        w11 = w_hh_ref[HH:, HH:]
        zb = (i - 1) % 2
        ha = h_ref[:, :HH]
        hb = h_ref[:, HH:]
        for k in range(CT):
            zk = z_ref[zb, k * B:(k + 1) * B, :]
            pre0 = zk[:, :HH] + _mm(ha, w00) + _mm(hb, w01)
            pre1 = zk[:, HH:] + _mm(ha, w10) + _mm(hb, w11)
            ha = jnp.tanh(pre0)
            hb = jnp.tanh(pre1)
            out_ref[:, k, :HH] = ha
            out_ref[:, k, HH:] = hb
        h_ref[:, :HH] = ha
        h_ref[:, HH:] = hb


@jax.jit
def kernel(actions, state_indices, emb, W_ih, W_hh, b_ih, b_hh):
    a_tm = jnp.swapaxes(actions, 0, 1).reshape(T * B, ACTION_DIM)
    idx_tm = jnp.swapaxes(state_indices, 0, 1).reshape(T * B, 1).astype(jnp.int32)

    last = NT - 1
    out = pl.pallas_call(
        _rnn_kernel,
        grid=(NT + 1,),
        in_specs=[
            pl.BlockSpec((CT * B, ACTION_DIM), lambda i: (jnp.minimum(i, last), 0)),
            pl.BlockSpec((CT * B, 1), lambda i: (jnp.minimum(i, last), 0)),
            pl.BlockSpec((STATE_DIM, EMBED_DIM), lambda i: (0, 0)),
            pl.BlockSpec((H_DIM, ACTION_DIM + EMBED_DIM), lambda i: (0, 0)),
            pl.BlockSpec((H_DIM, H_DIM), lambda i: (0, 0)),
            pl.BlockSpec((1, H_DIM), lambda i: (0, 0)),
            pl.BlockSpec((1, H_DIM), lambda i: (0, 0)),
        ],
        out_specs=pl.BlockSpec((B, CT, H_DIM),
                               lambda i: (0, jnp.maximum(i - 1, 0), 0)),
        out_shape=jax.ShapeDtypeStruct((B, T, H_DIM), jnp.float32),
        scratch_shapes=[pltpu.VMEM((B, H_DIM), jnp.float32),
                        pltpu.VMEM((2, CT * B, H_DIM), jnp.float32)],
    )(a_tm, idx_tm, emb, W_ih, W_hh.T,
      b_ih.reshape(1, H_DIM), b_hh.reshape(1, H_DIM))

    return out
